# Initial kernel scaffold; baseline (speedup 1.0000x reference)
import jax
import jax.numpy as jnp
from jax.experimental import pallas as pl


def kernel(x, edge_index, Ws0, Wn0, b0, Ws1, Wn1, b1, Ws2, Wn2, b2):
    raise NotImplementedError("write your pallas kernel here")


# speedup vs baseline: 6.8261x; 6.8261x over previous
import jax
import jax.numpy as jnp
from jax import lax
from jax.experimental import pallas as pl
from jax.experimental.pallas import tpu as pltpu
from jax.experimental.pallas import tpu_sc as plsc

N = 10000
E = 320000
D = 128
NC = 2
NS = 16
NW = NC * NS
EPW = E // NW
B = 80
NB = EPW // B
NCH = 5
BPC = NB // NCH
NP_ = 10240
RPT = NP_ // NS

_MESH = plsc.VectorSubcoreMesh(
    core_axis_name="c", subcore_axis_name="s", num_cores=NC, num_subcores=NS)


def _sc_agg_body(h_hbm, src_hbm, dst_hbm, z_hbm,
                 agg_out, src_v, dst_v, rows_v, sem, agg_sh):
  c = lax.axis_index("c")
  s = lax.axis_index("s")
  wid = s * NC + c
  rows = pl.ds(s * RPT, RPT)
  pltpu.sync_copy(z_hbm.at[rows], agg_sh.at[rows])
  plsc.subcore_barrier()

  def chunk(ch, carry):
    pltpu.sync_copy(src_hbm.at[wid, ch], src_v)
    pltpu.sync_copy(dst_hbm.at[wid, ch], dst_v)

    def step(j, carry):
      pltpu.async_copy(h_hbm.at[src_v.at[j]], rows_v, sem).wait()
      pltpu.sync_copy(rows_v, agg_sh.at[dst_v.at[j]], add=True)
      return carry

    return lax.fori_loop(0, BPC, step, carry)

  lax.fori_loop(0, NCH, chunk, 0)
  plsc.subcore_barrier()
  pltpu.sync_copy(agg_sh.at[rows], agg_out.at[c, rows])


_sc_agg = pl.kernel(
    _sc_agg_body,
    out_type=jax.ShapeDtypeStruct((NC, NP_, D), jnp.float32),
    mesh=_MESH,
    scratch_types=[
        pltpu.VMEM((BPC, B), jnp.int32),
        pltpu.VMEM((BPC, B), jnp.int32),
        pltpu.VMEM((B, D), jnp.float32),
        pltpu.SemaphoreType.DMA,
        pltpu.VMEM_SHARED((NP_, D), jnp.float32),
    ])


def _sc_deg_body(dst_hbm, z_hbm, ones_hbm,
                 deg_out, dst_v, ones_v, deg_sh):
  c = lax.axis_index("c")
  s = lax.axis_index("s")
  wid = s * NC + c
  rows = pl.ds(s * RPT, RPT)
  pltpu.sync_copy(z_hbm.at[rows], deg_sh.at[rows])
  pltpu.sync_copy(ones_hbm, ones_v)
  plsc.subcore_barrier()

  def chunk(ch, carry):
    pltpu.sync_copy(dst_hbm.at[wid, ch], dst_v)

    def step(j, carry):
      pltpu.sync_copy(ones_v, deg_sh.at[dst_v.at[j]], add=True)
      return carry

    return lax.fori_loop(0, BPC, step, carry)

  lax.fori_loop(0, NCH, chunk, 0)
  plsc.subcore_barrier()
  pltpu.sync_copy(deg_sh.at[rows], deg_out.at[c, rows])


_sc_deg = pl.kernel(
    _sc_deg_body,
    out_type=jax.ShapeDtypeStruct((NC, NP_, D), jnp.float32),
    mesh=_MESH,
    scratch_types=[
        pltpu.VMEM((BPC, B), jnp.int32),
        pltpu.VMEM((B, D), jnp.float32),
        pltpu.VMEM_SHARED((NP_, D), jnp.float32),
    ])


def _make_tc_layer(relu: bool):
  blk = 1000

  def body(h_ref, agg_ref, deg_ref, ws_ref, wn_ref, b_ref, o_ref):
    a = agg_ref[0] + agg_ref[1]
    dg = deg_ref[0, :, 0:1] + deg_ref[1, :, 0:1]
    h_neigh = a * (1.0 / jnp.maximum(dg, 1.0))
    out = (jnp.dot(h_ref[...], ws_ref[...], preferred_element_type=jnp.float32)
           + jnp.dot(h_neigh, wn_ref[...], preferred_element_type=jnp.float32)
           + b_ref[...])
    if relu:
      out = jnp.maximum(out, 0.0)
    o_ref[...] = out

  return pl.pallas_call(
      body,
      grid=(N // blk,),
      in_specs=[
          pl.BlockSpec((blk, D), lambda i: (i, 0)),
          pl.BlockSpec((NC, blk, D), lambda i: (0, i, 0)),
          pl.BlockSpec((NC, blk, D), lambda i: (0, i, 0)),
          pl.BlockSpec((D, D), lambda i: (0, 0)),
          pl.BlockSpec((D, D), lambda i: (0, 0)),
          pl.BlockSpec((1, D), lambda i: (0, 0)),
      ],
      out_specs=pl.BlockSpec((blk, D), lambda i: (i, 0)),
      out_shape=jax.ShapeDtypeStruct((N, D), jnp.float32),
  )


_tc_relu = _make_tc_layer(True)
_tc_last = _make_tc_layer(False)


def kernel(x, edge_index, Ws0, Wn0, b0, Ws1, Wn1, b1, Ws2, Wn2, b2):
  src = edge_index[0].reshape(NW, NCH, BPC, B)
  dst = edge_index[1].reshape(NW, NCH, BPC, B)
  z = jnp.zeros((NP_, D), jnp.float32)
  ones = jnp.ones((B, D), jnp.float32)

  deg = _sc_deg(dst, z, ones)
  agg = _sc_agg(x, src, dst, z)
  h = _tc_relu(x, agg, deg, Ws0, Wn0, b0.reshape(1, D))
  agg = _sc_agg(h, src, dst, z)
  h = _tc_relu(h, agg, deg, Ws1, Wn1, b1.reshape(1, D))
  agg = _sc_agg(h, src, dst, z)
  return _tc_last(h, agg, deg, Ws2, Wn2, b2.reshape(1, D))


# speedup vs baseline: 10.1141x; 1.4817x over previous
import jax
import jax.numpy as jnp
from jax import lax
from jax.experimental import pallas as pl
from jax.experimental.pallas import tpu as pltpu
from jax.experimental.pallas import tpu_sc as plsc

N = 10000
E = 320000
D = 128
NC = 2
NS = 16
NW = NC * NS
EPW = E // NW
B = 80
NB = EPW // B
NCH = 5
BPC = NB // NCH
NP_ = 10240
RPT = NP_ // NS

_MESH = plsc.VectorSubcoreMesh(
    core_axis_name="c", subcore_axis_name="s", num_cores=NC, num_subcores=NS)


def _sc_agg_body(h_hbm, src_hbm, dst_hbm, z_hbm,
                 agg_out, src_v, dst_v, buf_a, buf_b, sem, agg_sh):
  c = lax.axis_index("c")
  s = lax.axis_index("s")
  wid = s * NC + c
  rows = pl.ds(s * RPT, RPT)
  pltpu.sync_copy(z_hbm.at[rows], agg_sh.at[rows])
  plsc.subcore_barrier()

  def wait_gather(buf):
    pltpu.make_async_copy(h_hbm.at[pl.ds(0, B)], buf, sem).wait()

  def chunk(ch, carry):
    pltpu.sync_copy(src_hbm.at[wid, ch], src_v)
    pltpu.sync_copy(dst_hbm.at[wid, ch], dst_v)
    pltpu.async_copy(h_hbm.at[src_v.at[0]], buf_a, sem)

    def pair(p, carry):
      j0 = 2 * p
      pltpu.async_copy(h_hbm.at[src_v.at[j0 + 1]], buf_b, sem)
      wait_gather(buf_a)
      pltpu.sync_copy(buf_a, agg_sh.at[dst_v.at[j0]], add=True)
      pltpu.async_copy(h_hbm.at[src_v.at[j0 + 2]], buf_a, sem)
      wait_gather(buf_b)
      pltpu.sync_copy(buf_b, agg_sh.at[dst_v.at[j0 + 1]], add=True)
      return carry

    carry = lax.fori_loop(0, (BPC - 1) // 2, pair, carry)
    wait_gather(buf_a)
    pltpu.sync_copy(buf_a, agg_sh.at[dst_v.at[BPC - 1]], add=True)
    return carry

  lax.fori_loop(0, NCH, chunk, 0)
  plsc.subcore_barrier()
  pltpu.sync_copy(agg_sh.at[rows], agg_out.at[c, rows])


_sc_agg = pl.kernel(
    _sc_agg_body,
    out_type=jax.ShapeDtypeStruct((NC, NP_, D), jnp.float32),
    mesh=_MESH,
    scratch_types=[
        pltpu.VMEM((BPC, B), jnp.int32),
        pltpu.VMEM((BPC, B), jnp.int32),
        pltpu.VMEM((B, D), jnp.float32),
        pltpu.VMEM((B, D), jnp.float32),
        pltpu.SemaphoreType.DMA,
        pltpu.VMEM_SHARED((NP_, D), jnp.float32),
    ])


def _sc_deg_body(dst_hbm, z_hbm, ones_hbm,
                 deg_out, dst_v, ones_v, deg_sh):
  c = lax.axis_index("c")
  s = lax.axis_index("s")
  wid = s * NC + c
  rows = pl.ds(s * RPT, RPT)
  pltpu.sync_copy(z_hbm.at[rows], deg_sh.at[rows])
  pltpu.sync_copy(ones_hbm, ones_v)
  plsc.subcore_barrier()

  def chunk(ch, carry):
    pltpu.sync_copy(dst_hbm.at[wid, ch], dst_v)

    def step(j, carry):
      pltpu.sync_copy(ones_v, deg_sh.at[dst_v.at[j]], add=True)
      return carry

    return lax.fori_loop(0, BPC, step, carry)

  lax.fori_loop(0, NCH, chunk, 0)
  plsc.subcore_barrier()
  pltpu.sync_copy(deg_sh.at[rows], deg_out.at[c, rows])


_sc_deg = pl.kernel(
    _sc_deg_body,
    out_type=jax.ShapeDtypeStruct((NC, NP_, D), jnp.float32),
    mesh=_MESH,
    scratch_types=[
        pltpu.VMEM((BPC, B), jnp.int32),
        pltpu.VMEM((B, D), jnp.float32),
        pltpu.VMEM_SHARED((NP_, D), jnp.float32),
    ])


def _make_tc_layer(relu: bool):
  blk = 1000

  def body(h_ref, agg_ref, deg_ref, ws_ref, wn_ref, b_ref, o_ref):
    a = agg_ref[0] + agg_ref[1]
    dg = deg_ref[0, :, 0:1] + deg_ref[1, :, 0:1]
    h_neigh = a * (1.0 / jnp.maximum(dg, 1.0))
    out = (jnp.dot(h_ref[...], ws_ref[...], preferred_element_type=jnp.float32)
           + jnp.dot(h_neigh, wn_ref[...], preferred_element_type=jnp.float32)
           + b_ref[...])
    if relu:
      out = jnp.maximum(out, 0.0)
    o_ref[...] = out

  return pl.pallas_call(
      body,
      grid=(N // blk,),
      in_specs=[
          pl.BlockSpec((blk, D), lambda i: (i, 0)),
          pl.BlockSpec((NC, blk, D), lambda i: (0, i, 0)),
          pl.BlockSpec((NC, blk, D), lambda i: (0, i, 0)),
          pl.BlockSpec((D, D), lambda i: (0, 0)),
          pl.BlockSpec((D, D), lambda i: (0, 0)),
          pl.BlockSpec((1, D), lambda i: (0, 0)),
      ],
      out_specs=pl.BlockSpec((blk, D), lambda i: (i, 0)),
      out_shape=jax.ShapeDtypeStruct((N, D), jnp.float32),
  )


_tc_relu = _make_tc_layer(True)
_tc_last = _make_tc_layer(False)


def kernel(x, edge_index, Ws0, Wn0, b0, Ws1, Wn1, b1, Ws2, Wn2, b2):
  src = edge_index[0].reshape(NW, NCH, BPC, B)
  dst = edge_index[1].reshape(NW, NCH, BPC, B)
  z = jnp.zeros((NP_, D), jnp.float32)
  ones = jnp.ones((B, D), jnp.float32)

  deg = _sc_deg(dst, z, ones)
  agg = _sc_agg(x, src, dst, z)
  h = _tc_relu(x, agg, deg, Ws0, Wn0, b0.reshape(1, D))
  agg = _sc_agg(h, src, dst, z)
  h = _tc_relu(h, agg, deg, Ws1, Wn1, b1.reshape(1, D))
  agg = _sc_agg(h, src, dst, z)
  return _tc_last(h, agg, deg, Ws2, Wn2, b2.reshape(1, D))
